# baseline (speedup 1.0000x reference)
import functools

import jax
import jax.numpy as jnp
from jax import lax
from jax.experimental import pallas as pl
from jax.experimental.pallas import tpu as pltpu
from jax.experimental.pallas import tpu_sc as plsc

_LANES = 16
_NUM_WORKERS = 32
_CHUNK = 16384
_NBUF = 2
_UNROLL = 4

_COLS = 1024
_TC_BLOCK_ROWS = 512
_SC_SHARE = 10


def _sc_kernel_body(n_thr, per_worker, scores_hbm, thr_hbm, out_hbm,
                    thr_v, in0, in1, out0, out1,
                    si0, si1, so0, so1):
    in_b = (in0, in1)
    out_b = (out0, out1)
    in_sem = (si0, si1)
    out_sem = (so0, so1)

    wid = lax.axis_index("s") * 2 + lax.axis_index("c")
    base_off = wid * per_worker
    num_chunks = per_worker // _CHUNK

    pltpu.sync_copy(thr_hbm, thr_v)
    tb = [thr_v[pl.ds(j * _LANES, _LANES)] for j in range(n_thr)]

    for b in range(_NBUF):
        pltpu.async_copy(
            scores_hbm.at[pl.ds(base_off + b * _CHUNK, _CHUNK)],
            in_b[b], in_sem[b])

    @pl.loop(0, num_chunks, step=_NBUF)
    def _chunks(g0):
        for b in range(_NBUF):
            g = g0 + b
            off = base_off + g * _CHUNK
            pltpu.make_async_copy(
                scores_hbm.at[pl.ds(off, _CHUNK)], in_b[b], in_sem[b]).wait()

            @pl.when(g >= _NBUF)
            def _():
                pltpu.make_async_copy(
                    out_b[b],
                    out_hbm.at[pl.ds(off - _NBUF * _CHUNK, _CHUNK)],
                    out_sem[b]).wait()

            @pl.loop(0, _CHUNK // (_LANES * _UNROLL))
            def _vecs(i):
                for u in range(_UNROLL):
                    s = (i * _UNROLL + u) * _LANES
                    v = in_b[b][pl.ds(s, _LANES)]
                    acc = jnp.zeros((_LANES,), jnp.int32)
                    for j in range(n_thr):
                        acc = acc + jnp.where(v > tb[j], 1, 0)
                    out_b[b][pl.ds(s, _LANES)] = acc

            pltpu.async_copy(out_b[b], out_hbm.at[pl.ds(off, _CHUNK)],
                             out_sem[b])

            @pl.when(g + _NBUF < num_chunks)
            def _():
                pltpu.async_copy(
                    scores_hbm.at[pl.ds(off + _NBUF * _CHUNK, _CHUNK)],
                    in_b[b], in_sem[b])

    for b in range(_NBUF):
        last_off = base_off + (num_chunks - _NBUF + b) * _CHUNK
        pltpu.make_async_copy(
            out_b[b], out_hbm.at[pl.ds(last_off, _CHUNK)], out_sem[b]).wait()


def _tc_kernel_body(n_thr, thr_smem, x_ref, o_ref):
    x = x_ref[...]
    acc = jnp.zeros(x.shape, jnp.int32)
    for j in range(n_thr):
        acc = acc + (x > thr_smem[j]).astype(jnp.int32)
    o_ref[...] = acc


def _merge_body(tc_ref, sc_ref, o_ref):
    del tc_ref
    o_ref[...] = sc_ref[...]


_TC_BLK = 524288


def kernel(scores, thresholds):
    n = scores.shape[0]
    n_thr = thresholds.shape[0]
    n_sc = n * _SC_SHARE // 32
    assert n_sc % (_NUM_WORKERS * _CHUNK * _NBUF) == 0
    assert (n - n_sc) % _TC_BLK == 0 and n_sc % _TC_BLK == 0
    per_worker = n_sc // _NUM_WORKERS
    sc_blocks = n_sc // _TC_BLK

    tc_out = pl.pallas_call(
        functools.partial(_tc_kernel_body, n_thr),
        grid=((n - n_sc) // _TC_BLK,),
        in_specs=[
            pl.BlockSpec(memory_space=pltpu.SMEM),
            pl.BlockSpec((_TC_BLK,), lambda i: (sc_blocks + i,)),
        ],
        out_specs=pl.BlockSpec((_TC_BLK,), lambda i: (sc_blocks + i,)),
        out_shape=jax.ShapeDtypeStruct((n,), jnp.int32),
    )(thresholds.astype(jnp.float32), scores)

    thr_b = jnp.repeat(thresholds.astype(jnp.float32), _LANES)
    mesh = plsc.VectorSubcoreMesh(core_axis_name="c", subcore_axis_name="s")
    sc_fn = functools.partial(
        pl.kernel,
        out_type=jax.ShapeDtypeStruct((n_sc,), jnp.int32),
        mesh=mesh,
        compiler_params=pltpu.CompilerParams(needs_layout_passes=False),
        scratch_types=[
            pltpu.VMEM((n_thr * _LANES,), jnp.float32),
            pltpu.VMEM((_CHUNK,), jnp.float32),
            pltpu.VMEM((_CHUNK,), jnp.float32),
            pltpu.VMEM((_CHUNK,), jnp.int32),
            pltpu.VMEM((_CHUNK,), jnp.int32),
            pltpu.SemaphoreType.DMA,
            pltpu.SemaphoreType.DMA,
            pltpu.SemaphoreType.DMA,
            pltpu.SemaphoreType.DMA,
        ],
    )(functools.partial(_sc_kernel_body, n_thr, per_worker))
    sc_out = sc_fn(scores, thr_b)

    return pl.pallas_call(
        _merge_body,
        grid=(sc_blocks,),
        in_specs=[
            pl.BlockSpec(memory_space=pl.ANY),
            pl.BlockSpec((_TC_BLK,), lambda i: (i,)),
        ],
        out_specs=pl.BlockSpec((_TC_BLK,), lambda i: (i,)),
        out_shape=jax.ShapeDtypeStruct((n,), jnp.int32),
        input_output_aliases={0: 0},
    )(tc_out, sc_out)


# speedup vs baseline: 1.1210x; 1.1210x over previous
import functools

import jax
import jax.numpy as jnp
from jax import lax
from jax.experimental import pallas as pl
from jax.experimental.pallas import tpu as pltpu
from jax.experimental.pallas import tpu_sc as plsc

_LANES = 16
_NUM_WORKERS = 32
_CHUNK = 16384
_NBUF = 2
_UNROLL = 4

_COLS = 1024
_TC_BLOCK_ROWS = 512
_SC_SHARE = 8


def _sc_kernel_body(n_thr, per_worker, scores_hbm, thr_hbm, out_hbm,
                    thr_v, in0, in1, out0, out1,
                    si0, si1, so0, so1):
    in_b = (in0, in1)
    out_b = (out0, out1)
    in_sem = (si0, si1)
    out_sem = (so0, so1)

    wid = lax.axis_index("s") * 2 + lax.axis_index("c")
    base_off = wid * per_worker
    num_chunks = per_worker // _CHUNK

    pltpu.sync_copy(thr_hbm, thr_v)
    tb = [thr_v[pl.ds(j * _LANES, _LANES)] for j in range(n_thr)]

    for b in range(_NBUF):
        pltpu.async_copy(
            scores_hbm.at[pl.ds(base_off + b * _CHUNK, _CHUNK)],
            in_b[b], in_sem[b])

    @pl.loop(0, num_chunks, step=_NBUF)
    def _chunks(g0):
        for b in range(_NBUF):
            g = g0 + b
            off = base_off + g * _CHUNK
            pltpu.make_async_copy(
                scores_hbm.at[pl.ds(off, _CHUNK)], in_b[b], in_sem[b]).wait()

            @pl.when(g >= _NBUF)
            def _():
                pltpu.make_async_copy(
                    out_b[b],
                    out_hbm.at[pl.ds(off - _NBUF * _CHUNK, _CHUNK)],
                    out_sem[b]).wait()

            @pl.loop(0, _CHUNK // (_LANES * _UNROLL))
            def _vecs(i):
                for u in range(_UNROLL):
                    s = (i * _UNROLL + u) * _LANES
                    v = in_b[b][pl.ds(s, _LANES)]
                    acc = jnp.zeros((_LANES,), jnp.int32)
                    for j in range(n_thr):
                        acc = acc + jnp.where(v > tb[j], 1, 0)
                    out_b[b][pl.ds(s, _LANES)] = acc

            pltpu.async_copy(out_b[b], out_hbm.at[pl.ds(off, _CHUNK)],
                             out_sem[b])

            @pl.when(g + _NBUF < num_chunks)
            def _():
                pltpu.async_copy(
                    scores_hbm.at[pl.ds(off + _NBUF * _CHUNK, _CHUNK)],
                    in_b[b], in_sem[b])

    for b in range(_NBUF):
        last_off = base_off + (num_chunks - _NBUF + b) * _CHUNK
        pltpu.make_async_copy(
            out_b[b], out_hbm.at[pl.ds(last_off, _CHUNK)], out_sem[b]).wait()


def _tc_kernel_body(n_thr, thr_smem, x_ref, o_ref):
    x = x_ref[...]
    acc = jnp.zeros(x.shape, jnp.int32)
    for j in range(n_thr):
        acc = acc + (x > thr_smem[j]).astype(jnp.int32)
    o_ref[...] = acc


def _merge_body(tc_ref, sc_ref, o_ref):
    del tc_ref
    o_ref[...] = sc_ref[...]


_TC_BLK = 524288


def kernel(scores, thresholds):
    n = scores.shape[0]
    n_thr = thresholds.shape[0]
    n_sc = n * _SC_SHARE // 32
    assert n_sc % (_NUM_WORKERS * _CHUNK * _NBUF) == 0
    assert (n - n_sc) % _TC_BLK == 0 and n_sc % _TC_BLK == 0
    per_worker = n_sc // _NUM_WORKERS
    sc_blocks = n_sc // _TC_BLK

    tc_out = pl.pallas_call(
        functools.partial(_tc_kernel_body, n_thr),
        grid=((n - n_sc) // _TC_BLK,),
        in_specs=[
            pl.BlockSpec(memory_space=pltpu.SMEM),
            pl.BlockSpec((_TC_BLK,), lambda i: (sc_blocks + i,)),
        ],
        out_specs=pl.BlockSpec((_TC_BLK,), lambda i: (sc_blocks + i,)),
        out_shape=jax.ShapeDtypeStruct((n,), jnp.int32),
    )(thresholds.astype(jnp.float32), scores)

    thr_b = jnp.repeat(thresholds.astype(jnp.float32), _LANES)
    mesh = plsc.VectorSubcoreMesh(core_axis_name="c", subcore_axis_name="s")
    sc_fn = functools.partial(
        pl.kernel,
        out_type=jax.ShapeDtypeStruct((n_sc,), jnp.int32),
        mesh=mesh,
        compiler_params=pltpu.CompilerParams(needs_layout_passes=False),
        scratch_types=[
            pltpu.VMEM((n_thr * _LANES,), jnp.float32),
            pltpu.VMEM((_CHUNK,), jnp.float32),
            pltpu.VMEM((_CHUNK,), jnp.float32),
            pltpu.VMEM((_CHUNK,), jnp.int32),
            pltpu.VMEM((_CHUNK,), jnp.int32),
            pltpu.SemaphoreType.DMA,
            pltpu.SemaphoreType.DMA,
            pltpu.SemaphoreType.DMA,
            pltpu.SemaphoreType.DMA,
        ],
    )(functools.partial(_sc_kernel_body, n_thr, per_worker))
    sc_out = sc_fn(scores, thr_b)

    return pl.pallas_call(
        _merge_body,
        grid=(sc_blocks,),
        in_specs=[
            pl.BlockSpec(memory_space=pl.ANY),
            pl.BlockSpec((_TC_BLK,), lambda i: (i,)),
        ],
        out_specs=pl.BlockSpec((_TC_BLK,), lambda i: (i,)),
        out_shape=jax.ShapeDtypeStruct((n,), jnp.int32),
        input_output_aliases={0: 0},
    )(tc_out, sc_out)


# speedup vs baseline: 1.1911x; 1.0625x over previous
import functools

import jax
import jax.numpy as jnp
from jax import lax
from jax.experimental import pallas as pl
from jax.experimental.pallas import tpu as pltpu
from jax.experimental.pallas import tpu_sc as plsc

_LANES = 16
_NUM_WORKERS = 32
_CHUNK = 16384
_NBUF = 2
_UNROLL = 4

_COLS = 1024
_TC_BLOCK_ROWS = 512
_SC_SHARE = 8


def _sc_kernel_body(n_thr, per_worker, scores_hbm, thr_hbm, out_hbm,
                    thr_v, in0, in1, out0, out1,
                    si0, si1, so0, so1):
    in_b = (in0, in1)
    out_b = (out0, out1)
    in_sem = (si0, si1)
    out_sem = (so0, so1)

    wid = lax.axis_index("s") * 2 + lax.axis_index("c")
    base_off = wid * per_worker
    num_chunks = per_worker // _CHUNK

    pltpu.sync_copy(thr_hbm, thr_v)
    tb = [thr_v[pl.ds(j * _LANES, _LANES)] for j in range(n_thr)]

    for b in range(_NBUF):
        pltpu.async_copy(
            scores_hbm.at[pl.ds(base_off + b * _CHUNK, _CHUNK)],
            in_b[b], in_sem[b])

    @pl.loop(0, num_chunks, step=_NBUF)
    def _chunks(g0):
        for b in range(_NBUF):
            g = g0 + b
            off = base_off + g * _CHUNK
            pltpu.make_async_copy(
                scores_hbm.at[pl.ds(off, _CHUNK)], in_b[b], in_sem[b]).wait()

            @pl.when(g >= _NBUF)
            def _():
                pltpu.make_async_copy(
                    out_b[b],
                    out_hbm.at[pl.ds(off - _NBUF * _CHUNK, _CHUNK)],
                    out_sem[b]).wait()

            @pl.loop(0, _CHUNK // (_LANES * _UNROLL))
            def _vecs(i):
                for u in range(_UNROLL):
                    s = (i * _UNROLL + u) * _LANES
                    v = in_b[b][pl.ds(s, _LANES)]
                    acc = jnp.zeros((_LANES,), jnp.int32)
                    for j in range(n_thr):
                        acc = acc + jnp.where(v > tb[j], 1, 0)
                    out_b[b][pl.ds(s, _LANES)] = acc

            pltpu.async_copy(out_b[b], out_hbm.at[pl.ds(off, _CHUNK)],
                             out_sem[b])

            @pl.when(g + _NBUF < num_chunks)
            def _():
                pltpu.async_copy(
                    scores_hbm.at[pl.ds(off + _NBUF * _CHUNK, _CHUNK)],
                    in_b[b], in_sem[b])

    for b in range(_NBUF):
        last_off = base_off + (num_chunks - _NBUF + b) * _CHUNK
        pltpu.make_async_copy(
            out_b[b], out_hbm.at[pl.ds(last_off, _CHUNK)], out_sem[b]).wait()


def _tc_kernel_body(n_thr, thr_smem, x_ref, o_ref):
    x = x_ref[...]
    acc = jnp.zeros(x.shape, jnp.int32)
    for j in range(n_thr):
        acc = acc + (x > thr_smem[j]).astype(jnp.int32)
    o_ref[...] = acc


def _merge_body(tc_ref, sc_ref, o_ref):
    del tc_ref
    o_ref[...] = sc_ref[...]


_TC_BLK = 1048576


def kernel(scores, thresholds):
    n = scores.shape[0]
    n_thr = thresholds.shape[0]
    n_sc = n * _SC_SHARE // 32
    assert n_sc % (_NUM_WORKERS * _CHUNK * _NBUF) == 0
    assert (n - n_sc) % _TC_BLK == 0 and n_sc % _TC_BLK == 0
    per_worker = n_sc // _NUM_WORKERS
    sc_blocks = n_sc // _TC_BLK

    tc_out = pl.pallas_call(
        functools.partial(_tc_kernel_body, n_thr),
        grid=((n - n_sc) // _TC_BLK,),
        in_specs=[
            pl.BlockSpec(memory_space=pltpu.SMEM),
            pl.BlockSpec((_TC_BLK,), lambda i: (sc_blocks + i,)),
        ],
        out_specs=pl.BlockSpec((_TC_BLK,), lambda i: (sc_blocks + i,)),
        out_shape=jax.ShapeDtypeStruct((n,), jnp.int32),
    )(thresholds.astype(jnp.float32), scores)

    thr_b = jnp.repeat(thresholds.astype(jnp.float32), _LANES)
    mesh = plsc.VectorSubcoreMesh(core_axis_name="c", subcore_axis_name="s")
    sc_fn = functools.partial(
        pl.kernel,
        out_type=jax.ShapeDtypeStruct((n_sc,), jnp.int32),
        mesh=mesh,
        compiler_params=pltpu.CompilerParams(needs_layout_passes=False),
        scratch_types=[
            pltpu.VMEM((n_thr * _LANES,), jnp.float32),
            pltpu.VMEM((_CHUNK,), jnp.float32),
            pltpu.VMEM((_CHUNK,), jnp.float32),
            pltpu.VMEM((_CHUNK,), jnp.int32),
            pltpu.VMEM((_CHUNK,), jnp.int32),
            pltpu.SemaphoreType.DMA,
            pltpu.SemaphoreType.DMA,
            pltpu.SemaphoreType.DMA,
            pltpu.SemaphoreType.DMA,
        ],
    )(functools.partial(_sc_kernel_body, n_thr, per_worker))
    sc_out = sc_fn(scores, thr_b)

    return pl.pallas_call(
        _merge_body,
        grid=(sc_blocks,),
        in_specs=[
            pl.BlockSpec(memory_space=pl.ANY),
            pl.BlockSpec((_TC_BLK,), lambda i: (i,)),
        ],
        out_specs=pl.BlockSpec((_TC_BLK,), lambda i: (i,)),
        out_shape=jax.ShapeDtypeStruct((n,), jnp.int32),
        input_output_aliases={0: 0},
    )(tc_out, sc_out)


# speedup vs baseline: 1.1920x; 1.0008x over previous
import functools

import jax
import jax.numpy as jnp
from jax import lax
from jax.experimental import pallas as pl
from jax.experimental.pallas import tpu as pltpu
from jax.experimental.pallas import tpu_sc as plsc

_LANES = 16
_NUM_WORKERS = 32
_CHUNK = 16384
_NBUF = 2
_UNROLL = 4

_COLS = 1024
_TC_BLOCK_ROWS = 512
_SC_SHARE = 8


def _sc_kernel_body(n_thr, per_worker, scores_hbm, thr_hbm, out_hbm,
                    thr_v, in0, in1, out0, out1,
                    si0, si1, so0, so1):
    in_b = (in0, in1)
    out_b = (out0, out1)
    in_sem = (si0, si1)
    out_sem = (so0, so1)

    wid = lax.axis_index("s") * 2 + lax.axis_index("c")
    base_off = wid * per_worker
    num_chunks = per_worker // _CHUNK

    pltpu.sync_copy(thr_hbm, thr_v)
    tb = [thr_v[pl.ds(j * _LANES, _LANES)] for j in range(n_thr)]

    for b in range(_NBUF):
        pltpu.async_copy(
            scores_hbm.at[pl.ds(base_off + b * _CHUNK, _CHUNK)],
            in_b[b], in_sem[b])

    @pl.loop(0, num_chunks, step=_NBUF)
    def _chunks(g0):
        for b in range(_NBUF):
            g = g0 + b
            off = base_off + g * _CHUNK
            pltpu.make_async_copy(
                scores_hbm.at[pl.ds(off, _CHUNK)], in_b[b], in_sem[b]).wait()

            @pl.when(g >= _NBUF)
            def _():
                pltpu.make_async_copy(
                    out_b[b],
                    out_hbm.at[pl.ds(off - _NBUF * _CHUNK, _CHUNK)],
                    out_sem[b]).wait()

            @pl.loop(0, _CHUNK // (_LANES * _UNROLL))
            def _vecs(i):
                for u in range(_UNROLL):
                    s = (i * _UNROLL + u) * _LANES
                    v = in_b[b][pl.ds(s, _LANES)]
                    acc = jnp.zeros((_LANES,), jnp.int32)
                    for j in range(n_thr):
                        acc = acc + jnp.where(v > tb[j], 1, 0)
                    out_b[b][pl.ds(s, _LANES)] = acc

            pltpu.async_copy(out_b[b], out_hbm.at[pl.ds(off, _CHUNK)],
                             out_sem[b])

            @pl.when(g + _NBUF < num_chunks)
            def _():
                pltpu.async_copy(
                    scores_hbm.at[pl.ds(off + _NBUF * _CHUNK, _CHUNK)],
                    in_b[b], in_sem[b])

    for b in range(_NBUF):
        last_off = base_off + (num_chunks - _NBUF + b) * _CHUNK
        pltpu.make_async_copy(
            out_b[b], out_hbm.at[pl.ds(last_off, _CHUNK)], out_sem[b]).wait()


def _tc_kernel_body(n_thr, thr_smem, x_ref, o_ref):
    x = x_ref[...]
    acc = jnp.zeros(x.shape, jnp.int32)
    for j in range(n_thr):
        acc = acc + (x > thr_smem[j]).astype(jnp.int32)
    o_ref[...] = acc


def _merge_body(tc_ref, sc_ref, o_ref):
    del tc_ref
    o_ref[...] = sc_ref[...]


_TC_BLK = 2097152


def kernel(scores, thresholds):
    n = scores.shape[0]
    n_thr = thresholds.shape[0]
    n_sc = n * _SC_SHARE // 32
    assert n_sc % (_NUM_WORKERS * _CHUNK * _NBUF) == 0
    assert (n - n_sc) % _TC_BLK == 0 and n_sc % _TC_BLK == 0
    per_worker = n_sc // _NUM_WORKERS
    sc_blocks = n_sc // _TC_BLK

    tc_out = pl.pallas_call(
        functools.partial(_tc_kernel_body, n_thr),
        grid=((n - n_sc) // _TC_BLK,),
        in_specs=[
            pl.BlockSpec(memory_space=pltpu.SMEM),
            pl.BlockSpec((_TC_BLK,), lambda i: (sc_blocks + i,)),
        ],
        out_specs=pl.BlockSpec((_TC_BLK,), lambda i: (sc_blocks + i,)),
        out_shape=jax.ShapeDtypeStruct((n,), jnp.int32),
    )(thresholds.astype(jnp.float32), scores)

    thr_b = jnp.repeat(thresholds.astype(jnp.float32), _LANES)
    mesh = plsc.VectorSubcoreMesh(core_axis_name="c", subcore_axis_name="s")
    sc_fn = functools.partial(
        pl.kernel,
        out_type=jax.ShapeDtypeStruct((n_sc,), jnp.int32),
        mesh=mesh,
        compiler_params=pltpu.CompilerParams(needs_layout_passes=False),
        scratch_types=[
            pltpu.VMEM((n_thr * _LANES,), jnp.float32),
            pltpu.VMEM((_CHUNK,), jnp.float32),
            pltpu.VMEM((_CHUNK,), jnp.float32),
            pltpu.VMEM((_CHUNK,), jnp.int32),
            pltpu.VMEM((_CHUNK,), jnp.int32),
            pltpu.SemaphoreType.DMA,
            pltpu.SemaphoreType.DMA,
            pltpu.SemaphoreType.DMA,
            pltpu.SemaphoreType.DMA,
        ],
    )(functools.partial(_sc_kernel_body, n_thr, per_worker))
    sc_out = sc_fn(scores, thr_b)

    return pl.pallas_call(
        _merge_body,
        grid=(sc_blocks,),
        in_specs=[
            pl.BlockSpec(memory_space=pl.ANY),
            pl.BlockSpec((_TC_BLK,), lambda i: (i,)),
        ],
        out_specs=pl.BlockSpec((_TC_BLK,), lambda i: (i,)),
        out_shape=jax.ShapeDtypeStruct((n,), jnp.int32),
        input_output_aliases={0: 0},
    )(tc_out, sc_out)


# speedup vs baseline: 1.1932x; 1.0010x over previous
import functools

import jax
import jax.numpy as jnp
from jax import lax
from jax.experimental import pallas as pl
from jax.experimental.pallas import tpu as pltpu
from jax.experimental.pallas import tpu_sc as plsc

_LANES = 16
_NUM_WORKERS = 32
_CHUNK = 16384
_NBUF = 2
_UNROLL = 4

_SC_SHARE = 8


def _sc_kernel_body(n_thr, per_worker, scores_hbm, thr_hbm, out_hbm,
                    thr_v, in0, in1, out0, out1,
                    si0, si1, so0, so1):
    in_b = (in0, in1)
    out_b = (out0, out1)
    in_sem = (si0, si1)
    out_sem = (so0, so1)

    wid = lax.axis_index("s") * 2 + lax.axis_index("c")
    base_off = wid * per_worker
    num_chunks = per_worker // _CHUNK

    pltpu.sync_copy(thr_hbm, thr_v)
    tb = [thr_v[pl.ds(j * _LANES, _LANES)] for j in range(n_thr)]

    for b in range(_NBUF):
        pltpu.async_copy(
            scores_hbm.at[pl.ds(base_off + b * _CHUNK, _CHUNK)],
            in_b[b], in_sem[b])

    @pl.loop(0, num_chunks, step=_NBUF)
    def _chunks(g0):
        for b in range(_NBUF):
            g = g0 + b
            off = base_off + g * _CHUNK
            pltpu.make_async_copy(
                scores_hbm.at[pl.ds(off, _CHUNK)], in_b[b], in_sem[b]).wait()

            @pl.when(g >= _NBUF)
            def _():
                pltpu.make_async_copy(
                    out_b[b],
                    out_hbm.at[pl.ds(off - _NBUF * _CHUNK, _CHUNK)],
                    out_sem[b]).wait()

            @pl.loop(0, _CHUNK // (_LANES * _UNROLL))
            def _vecs(i):
                for u in range(_UNROLL):
                    s = (i * _UNROLL + u) * _LANES
                    v = in_b[b][pl.ds(s, _LANES)]
                    acc = jnp.zeros((_LANES,), jnp.int32)
                    for j in range(n_thr):
                        acc = acc + jnp.where(v > tb[j], 1, 0)
                    out_b[b][pl.ds(s, _LANES)] = acc

            pltpu.async_copy(out_b[b], out_hbm.at[pl.ds(off, _CHUNK)],
                             out_sem[b])

            @pl.when(g + _NBUF < num_chunks)
            def _():
                pltpu.async_copy(
                    scores_hbm.at[pl.ds(off + _NBUF * _CHUNK, _CHUNK)],
                    in_b[b], in_sem[b])

    for b in range(_NBUF):
        last_off = base_off + (num_chunks - _NBUF + b) * _CHUNK
        pltpu.make_async_copy(
            out_b[b], out_hbm.at[pl.ds(last_off, _CHUNK)], out_sem[b]).wait()


def _tc_kernel_body(n_thr, thr_smem, x_ref, o_ref):
    x = x_ref[...]
    acc = jnp.zeros(x.shape, jnp.int32)
    for j in range(n_thr):
        acc = acc + (x > thr_smem[j]).astype(jnp.int32)
    o_ref[...] = acc


def _merge_body(tc_ref, sc_ref, o_ref):
    del tc_ref
    o_ref[...] = sc_ref[...]


_TC_BLK = 2097152


def kernel(scores, thresholds):
    n = scores.shape[0]
    n_thr = thresholds.shape[0]
    n_sc = n * _SC_SHARE // 32
    assert n_sc % (_NUM_WORKERS * _CHUNK * _NBUF) == 0
    assert (n - n_sc) % _TC_BLK == 0 and n_sc % _TC_BLK == 0
    per_worker = n_sc // _NUM_WORKERS
    sc_blocks = n_sc // _TC_BLK

    tc_out = pl.pallas_call(
        functools.partial(_tc_kernel_body, n_thr),
        grid=((n - n_sc) // _TC_BLK,),
        in_specs=[
            pl.BlockSpec(memory_space=pltpu.SMEM),
            pl.BlockSpec((_TC_BLK,), lambda i: (sc_blocks + i,)),
        ],
        out_specs=pl.BlockSpec((_TC_BLK,), lambda i: (sc_blocks + i,)),
        out_shape=jax.ShapeDtypeStruct((n,), jnp.int32),
    )(thresholds.astype(jnp.float32), scores)

    thr_b = jnp.repeat(thresholds.astype(jnp.float32), _LANES)
    mesh = plsc.VectorSubcoreMesh(core_axis_name="c", subcore_axis_name="s")
    sc_fn = functools.partial(
        pl.kernel,
        out_type=jax.ShapeDtypeStruct((n_sc,), jnp.int32),
        mesh=mesh,
        compiler_params=pltpu.CompilerParams(needs_layout_passes=False),
        scratch_types=[
            pltpu.VMEM((n_thr * _LANES,), jnp.float32),
            pltpu.VMEM((_CHUNK,), jnp.float32),
            pltpu.VMEM((_CHUNK,), jnp.float32),
            pltpu.VMEM((_CHUNK,), jnp.int32),
            pltpu.VMEM((_CHUNK,), jnp.int32),
            pltpu.SemaphoreType.DMA,
            pltpu.SemaphoreType.DMA,
            pltpu.SemaphoreType.DMA,
            pltpu.SemaphoreType.DMA,
        ],
    )(functools.partial(_sc_kernel_body, n_thr, per_worker))
    sc_out = sc_fn(scores, thr_b)

    return pl.pallas_call(
        _merge_body,
        grid=(sc_blocks,),
        in_specs=[
            pl.BlockSpec(memory_space=pl.ANY),
            pl.BlockSpec((_TC_BLK,), lambda i: (i,)),
        ],
        out_specs=pl.BlockSpec((_TC_BLK,), lambda i: (i,)),
        out_shape=jax.ShapeDtypeStruct((n,), jnp.int32),
        input_output_aliases={0: 0},
    )(tc_out, sc_out)
